# baseline (speedup 1.0000x reference)
import dataclasses
import functools

import jax
import jax.numpy as jnp
from jax import lax
from jax.experimental import pallas as pl
from jax.experimental.pallas import tpu as pltpu
from jax.experimental.pallas import tpu_sc as plsc

_NUM_CORES = 2
_NUM_SUBCORES = 16
_NTILES = _NUM_CORES * _NUM_SUBCORES
_CH = 128


def _build_propagate(n, h, chunks_per_tile):
    rows_per_tile = (n // _NUM_SUBCORES) // 8 * 8
    rows_last = n - rows_per_tile * (_NUM_SUBCORES - 1)
    mesh = plsc.VectorSubcoreMesh(
        core_axis_name="c", subcore_axis_name="s", num_cores=_NUM_CORES,
        num_subcores=_NUM_SUBCORES)
    cp = pltpu.CompilerParams()
    if "needs_layout_passes" in pltpu.CompilerParams.__dataclass_fields__:
        cp = dataclasses.replace(cp, needs_layout_passes=False)

    @functools.partial(
        pl.kernel,
        out_type=jax.ShapeDtypeStruct((_NUM_CORES, n, h), jnp.float32),
        mesh=mesh,
        scratch_types=[
            pltpu.VMEM_SHARED((n, h), jnp.float32),
            pltpu.VMEM((4, _CH), jnp.int32),
            pltpu.VMEM((4, _CH), jnp.int32),
            pltpu.VMEM((4, _CH), jnp.float32),
            pltpu.VMEM((_CH, h), jnp.float32),
            pltpu.VMEM((_CH, h), jnp.float32),
            pltpu.SemaphoreType.DMA,
            pltpu.SemaphoreType.DMA,
            pltpu.SemaphoreType.DMA,
            pltpu.SemaphoreType.DMA,
            pltpu.SemaphoreType.DMA,
            pltpu.SemaphoreType.DMA,
            pltpu.SemaphoreType.DMA,
            pltpu.SemaphoreType.DMA,
        ],
        compiler_params=cp,
    )
    def propagate_sc(hx_hbm, src_hbm, dst_hbm, w_hbm, zinit_hbm, part_hbm,
                     acc_sh, sb, db, wb, rows0, rows1,
                     g0, g1, s0, s1, e0, e1, e2, e3):
        ci = lax.axis_index("c")
        si = lax.axis_index("s")
        tile = ci * _NUM_SUBCORES + si
        chunk0 = tile * chunks_per_tile

        r0 = pl.multiple_of(si * rows_per_tile, 8)

        @pl.when(si < _NUM_SUBCORES - 1)
        def _():
            pltpu.sync_copy(zinit_hbm.at[pl.ds(r0, rows_per_tile)],
                            acc_sh.at[pl.ds(r0, rows_per_tile)])

        @pl.when(si == _NUM_SUBCORES - 1)
        def _():
            pltpu.sync_copy(zinit_hbm.at[pl.ds(r0, rows_last)],
                            acc_sh.at[pl.ds(r0, rows_last)])

        plsc.subcore_barrier()

        dnums = lax.GatherDimensionNumbers(
            offset_dims=(), collapsed_slice_dims=(0,), start_index_map=(0,))

        def splat(vec, e):
            idx = jnp.full((16, 1), e, jnp.int32)
            return lax.gather(vec, idx, dnums, slice_sizes=(1,),
                              mode=lax.GatherScatterMode.PROMISE_IN_BOUNDS)

        def scale(rbuf, p):
            @pl.loop(0, _CH // 16)
            def _grp(gi):
                wv16 = wb[p, pl.ds(gi * 16, 16)]
                for e in range(16):
                    ei = gi * 16 + e
                    wsp = splat(wv16, e)
                    r = rbuf.at[ei]
                    for j in range(h // 16):
                        r[pl.ds(j * 16, 16)] = r[pl.ds(j * 16, 16)] * wsp

        rows = (rows0, rows1)
        gsem = (g0, g1)
        ssem = (s0, s1)
        esem = (e0, e1, e2, e3)
        nc = chunks_per_tile

        def eslice(arr, t):
            off = pl.multiple_of((chunk0 + t) * _CH, 8)
            return arr.at[pl.ds(off, _CH)]

        def start_eloads(t, p):
            pltpu.async_copy(eslice(src_hbm, t), sb.at[p], esem[p])
            pltpu.async_copy(eslice(dst_hbm, t), db.at[p], esem[p])
            pltpu.async_copy(eslice(w_hbm, t), wb.at[p], esem[p])

        def wait_eloads(t, p):
            pltpu.make_async_copy(eslice(src_hbm, t), sb.at[p], esem[p]).wait()
            pltpu.make_async_copy(eslice(dst_hbm, t), db.at[p], esem[p]).wait()
            pltpu.make_async_copy(eslice(w_hbm, t), wb.at[p], esem[p]).wait()

        def start_gather(p, p4):
            pltpu.async_copy(hx_hbm.at[sb.at[p4]], rows[p], gsem[p])

        def wait_gather(p, p4):
            pltpu.make_async_copy(hx_hbm.at[sb.at[p4]], rows[p], gsem[p]).wait()

        def start_scatter(p, p4):
            pltpu.async_copy(rows[p], acc_sh.at[db.at[p4]], ssem[p], add=True)

        def wait_scatter(p, p4):
            pltpu.make_async_copy(rows[p], acc_sh.at[db.at[p4]],
                                  ssem[p]).wait()

        start_eloads(0, 0)
        start_eloads(1, 1)
        wait_eloads(0, 0)
        start_gather(0, 0)

        @pl.loop(0, nc, step=4)
        def _quad(t):
            for i in range(4):
                c = t + i
                p = i % 2
                wait_gather(p, i)
                scale(rows[p], i)

                @pl.when(c >= 1)
                def _():
                    wait_scatter(1 - p, (i + 3) % 4)

                @pl.when(c + 1 < nc)
                def _():
                    wait_eloads(c + 1, (i + 1) % 4)
                    start_gather(1 - p, (i + 1) % 4)

                start_scatter(p, i)

                @pl.when(c + 2 < nc)
                def _():
                    start_eloads(c + 2, (i + 2) % 4)

        wait_scatter((nc - 1) % 2, (nc - 1) % 4)

        plsc.subcore_barrier()

        @pl.when(si < _NUM_SUBCORES - 1)
        def _():
            pltpu.sync_copy(acc_sh.at[pl.ds(r0, rows_per_tile)],
                            part_hbm.at[ci].at[pl.ds(r0, rows_per_tile)])

        @pl.when(si == _NUM_SUBCORES - 1)
        def _():
            pltpu.sync_copy(acc_sh.at[pl.ds(r0, rows_last)],
                            part_hbm.at[ci].at[pl.ds(r0, rows_last)])

    return propagate_sc


def _combine_first(parts, n, h):
    def body(a_ref, o_ref):
        o_ref[...] = a_ref[0] + a_ref[1]
    return pl.pallas_call(
        body, out_shape=jax.ShapeDtypeStruct((n, h), jnp.float32))(parts)


def _combine_step(parts, tprev2, n, h):
    def body(a_ref, t_ref, o_ref):
        o_ref[...] = 2.0 * (a_ref[0] + a_ref[1]) - t_ref[...]
    return pl.pallas_call(
        body, out_shape=jax.ShapeDtypeStruct((n, h), jnp.float32))(parts, tprev2)


def _final_tail(parts_last, ts, coeff_rows, gs_row, gb_row, nw_row, n, h):
    kmax = coeff_rows.shape[0] - 1
    eps = jnp.finfo(jnp.float32).eps

    def body(*refs):
        t_refs = refs[:kmax]
        a_ref = refs[kmax]
        coef_ref = refs[kmax + 1]
        gs_ref, gb_ref, nw_ref = refs[kmax + 2:kmax + 5]
        o_ref = refs[kmax + 5]
        t_last = 2.0 * (a_ref[0] + a_ref[1]) - t_refs[kmax - 2][...]
        res = coef_ref[kmax:kmax + 1] * t_last
        for k in range(kmax):
            res = res + coef_ref[k:k + 1] * t_refs[k][...]
        res = res * gs_ref[...] + gb_ref[...]
        ms = jnp.mean(res * res, axis=-1, keepdims=True)
        y = res * lax.rsqrt(ms + eps) * nw_ref[...]
        o_ref[...] = y * jax.nn.sigmoid(y)

    return pl.pallas_call(
        body, out_shape=jax.ShapeDtypeStruct((n, h), jnp.float32))(
            *ts, parts_last, coeff_rows, gs_row, gb_row, nw_row)


def kernel(x, edge_index, edge_weight_norm, cheb_coeffs, group_scale,
           group_bias, norm_weight):
    n, h = x.shape
    e = edge_index.shape[1]
    g = group_scale.shape[0]
    c = h // g
    kmax = cheb_coeffs.shape[1] - 1

    chunks_per_tile = -(-e // (_CH * _NTILES))
    chunks_per_tile = -(-chunks_per_tile // 8) * 8
    e_pad = chunks_per_tile * _CH * _NTILES
    pad = e_pad - e

    pad_idx = jnp.arange(pad, dtype=jnp.int32) % n
    src_p = jnp.concatenate([edge_index[0], pad_idx])
    dst_p = jnp.concatenate([edge_index[1], pad_idx])
    w_p = jnp.concatenate([edge_weight_norm, jnp.zeros((pad,), jnp.float32)])
    zinit = jnp.zeros((n, h), jnp.float32)

    coeff_rows = jnp.repeat(cheb_coeffs, c, axis=0).T
    gs_row = jnp.repeat(group_scale, c).reshape(1, h)
    gb_row = jnp.repeat(group_bias, c).reshape(1, h)
    nw_row = norm_weight.reshape(1, h)

    propagate_sc = _build_propagate(n, h, chunks_per_tile)

    def propagate(hx):
        return propagate_sc(hx, src_p, dst_p, w_p, zinit)

    parts = propagate(x)
    t1 = _combine_first(parts, n, h)
    ts = [x, t1]
    tprev2, tprev1 = x, t1
    for _k in range(2, kmax):
        parts = propagate(tprev1)
        tk = _combine_step(parts, tprev2, n, h)
        ts.append(tk)
        tprev2, tprev1 = tprev1, tk
    parts_last = propagate(tprev1)
    return _final_tail(parts_last, ts, coeff_rows, gs_row, gb_row, nw_row,
                       n, h)


# speedup vs baseline: 1.3017x; 1.3017x over previous
import dataclasses
import functools

import jax
import jax.numpy as jnp
from jax import lax
from jax.experimental import pallas as pl
from jax.experimental.pallas import tpu as pltpu
from jax.experimental.pallas import tpu_sc as plsc

_NUM_CORES = 2
_NUM_SUBCORES = 16
_NTILES = _NUM_CORES * _NUM_SUBCORES
_CH = 128


def _build_propagate(n, h, chunks_per_tile):
    rows_per_tile = (n // _NUM_SUBCORES) // 8 * 8
    rows_last = n - rows_per_tile * (_NUM_SUBCORES - 1)
    mesh = plsc.VectorSubcoreMesh(
        core_axis_name="c", subcore_axis_name="s", num_cores=_NUM_CORES,
        num_subcores=_NUM_SUBCORES)
    cp = pltpu.CompilerParams()
    if "needs_layout_passes" in pltpu.CompilerParams.__dataclass_fields__:
        cp = dataclasses.replace(cp, needs_layout_passes=False)

    @functools.partial(
        pl.kernel,
        out_type=jax.ShapeDtypeStruct((_NUM_CORES, n, h), jnp.float32),
        mesh=mesh,
        scratch_types=[
            pltpu.VMEM_SHARED((n, h), jnp.float32),
            pltpu.VMEM((4, _CH), jnp.int32),
            pltpu.VMEM((4, _CH), jnp.int32),
            pltpu.VMEM((4, _CH), jnp.float32),
            pltpu.VMEM((_CH, h), jnp.float32),
            pltpu.VMEM((_CH, h), jnp.float32),
            pltpu.SemaphoreType.DMA,
            pltpu.SemaphoreType.DMA,
            pltpu.SemaphoreType.DMA,
            pltpu.SemaphoreType.DMA,
            pltpu.SemaphoreType.DMA,
            pltpu.SemaphoreType.DMA,
            pltpu.SemaphoreType.DMA,
            pltpu.SemaphoreType.DMA,
        ],
        compiler_params=cp,
    )
    def propagate_sc(hx_hbm, src_hbm, dst_hbm, w_hbm, zinit_hbm, part_hbm,
                     acc_sh, sb, db, wb, rows0, rows1,
                     g0, g1, s0, s1, e0, e1, e2, e3):
        ci = lax.axis_index("c")
        si = lax.axis_index("s")
        tile = ci * _NUM_SUBCORES + si
        chunk0 = tile * chunks_per_tile

        r0 = pl.multiple_of(si * rows_per_tile, 8)

        @pl.when(si < _NUM_SUBCORES - 1)
        def _():
            pltpu.sync_copy(zinit_hbm.at[pl.ds(r0, rows_per_tile)],
                            acc_sh.at[pl.ds(r0, rows_per_tile)])

        @pl.when(si == _NUM_SUBCORES - 1)
        def _():
            pltpu.sync_copy(zinit_hbm.at[pl.ds(r0, rows_last)],
                            acc_sh.at[pl.ds(r0, rows_last)])

        plsc.subcore_barrier()

        dnums = lax.GatherDimensionNumbers(
            offset_dims=(), collapsed_slice_dims=(0,), start_index_map=(0,))

        def splat(vec, e):
            idx = jnp.full((16, 1), e, jnp.int32)
            return lax.gather(vec, idx, dnums, slice_sizes=(1,),
                              mode=lax.GatherScatterMode.PROMISE_IN_BOUNDS)

        def scale(rbuf, p):
            @pl.loop(0, _CH // 16)
            def _grp(gi):
                wv16 = wb[p, pl.ds(gi * 16, 16)]
                for e in range(16):
                    ei = gi * 16 + e
                    wsp = splat(wv16, e)
                    r = rbuf.at[ei]
                    for j in range(h // 16):
                        r[pl.ds(j * 16, 16)] = r[pl.ds(j * 16, 16)] * wsp

        rows = (rows0, rows1)
        gsem = (g0, g1)
        ssem = (s0, s1)
        esem = (e0, e1, e2, e3)
        nc = chunks_per_tile

        def eslice(arr, t):
            off = pl.multiple_of((chunk0 + t) * _CH, 8)
            return arr.at[pl.ds(off, _CH)]

        def start_eloads(t, p):
            pltpu.async_copy(eslice(src_hbm, t), sb.at[p], esem[p])
            pltpu.async_copy(eslice(dst_hbm, t), db.at[p], esem[p])
            pltpu.async_copy(eslice(w_hbm, t), wb.at[p], esem[p])

        def wait_eloads(t, p):
            pltpu.make_async_copy(eslice(src_hbm, t), sb.at[p], esem[p]).wait()
            pltpu.make_async_copy(eslice(dst_hbm, t), db.at[p], esem[p]).wait()
            pltpu.make_async_copy(eslice(w_hbm, t), wb.at[p], esem[p]).wait()

        def start_gather(p, p4):
            pltpu.async_copy(hx_hbm.at[sb.at[p4]], rows[p], gsem[p])

        def wait_gather(p, p4):
            pltpu.make_async_copy(hx_hbm.at[sb.at[p4]], rows[p], gsem[p]).wait()

        def start_scatter(p, p4):
            pltpu.async_copy(rows[p], acc_sh.at[db.at[p4]], ssem[p], add=True)

        def wait_scatter(p, p4):
            pltpu.make_async_copy(rows[p], acc_sh.at[db.at[p4]],
                                  ssem[p]).wait()

        start_eloads(0, 0)
        start_eloads(1, 1)
        wait_eloads(0, 0)
        start_gather(0, 0)

        @pl.loop(0, nc, step=4)
        def _quad(t):
            for i in range(4):
                c = t + i
                p = i % 2

                @pl.when(c >= 1)
                def _():
                    wait_scatter(1 - p, (i + 3) % 4)

                @pl.when(c + 1 < nc)
                def _():
                    wait_eloads(c + 1, (i + 1) % 4)
                    start_gather(1 - p, (i + 1) % 4)

                wait_gather(p, i)
                scale(rows[p], i)
                start_scatter(p, i)

                @pl.when(c + 2 < nc)
                def _():
                    start_eloads(c + 2, (i + 2) % 4)

        wait_scatter((nc - 1) % 2, (nc - 1) % 4)

        plsc.subcore_barrier()

        @pl.when(si < _NUM_SUBCORES - 1)
        def _():
            pltpu.sync_copy(acc_sh.at[pl.ds(r0, rows_per_tile)],
                            part_hbm.at[ci].at[pl.ds(r0, rows_per_tile)])

        @pl.when(si == _NUM_SUBCORES - 1)
        def _():
            pltpu.sync_copy(acc_sh.at[pl.ds(r0, rows_last)],
                            part_hbm.at[ci].at[pl.ds(r0, rows_last)])

    return propagate_sc


def _combine_first(parts, n, h):
    def body(a_ref, o_ref):
        o_ref[...] = a_ref[0] + a_ref[1]
    return pl.pallas_call(
        body, out_shape=jax.ShapeDtypeStruct((n, h), jnp.float32))(parts)


def _combine_step(parts, tprev2, n, h):
    def body(a_ref, t_ref, o_ref):
        o_ref[...] = 2.0 * (a_ref[0] + a_ref[1]) - t_ref[...]
    return pl.pallas_call(
        body, out_shape=jax.ShapeDtypeStruct((n, h), jnp.float32))(parts, tprev2)


def _final_tail(parts_last, ts, coeff_rows, gs_row, gb_row, nw_row, n, h):
    kmax = coeff_rows.shape[0] - 1
    eps = jnp.finfo(jnp.float32).eps

    def body(*refs):
        t_refs = refs[:kmax]
        a_ref = refs[kmax]
        coef_ref = refs[kmax + 1]
        gs_ref, gb_ref, nw_ref = refs[kmax + 2:kmax + 5]
        o_ref = refs[kmax + 5]
        t_last = 2.0 * (a_ref[0] + a_ref[1]) - t_refs[kmax - 2][...]
        res = coef_ref[kmax:kmax + 1] * t_last
        for k in range(kmax):
            res = res + coef_ref[k:k + 1] * t_refs[k][...]
        res = res * gs_ref[...] + gb_ref[...]
        ms = jnp.mean(res * res, axis=-1, keepdims=True)
        y = res * lax.rsqrt(ms + eps) * nw_ref[...]
        o_ref[...] = y * jax.nn.sigmoid(y)

    return pl.pallas_call(
        body, out_shape=jax.ShapeDtypeStruct((n, h), jnp.float32))(
            *ts, parts_last, coeff_rows, gs_row, gb_row, nw_row)


def kernel(x, edge_index, edge_weight_norm, cheb_coeffs, group_scale,
           group_bias, norm_weight):
    n, h = x.shape
    e = edge_index.shape[1]
    g = group_scale.shape[0]
    c = h // g
    kmax = cheb_coeffs.shape[1] - 1

    chunks_per_tile = -(-e // (_CH * _NTILES))
    chunks_per_tile = -(-chunks_per_tile // 8) * 8
    e_pad = chunks_per_tile * _CH * _NTILES
    pad = e_pad - e

    pad_idx = jnp.arange(pad, dtype=jnp.int32) % n
    src_p = jnp.concatenate([edge_index[0], pad_idx])
    dst_p = jnp.concatenate([edge_index[1], pad_idx])
    w_p = jnp.concatenate([edge_weight_norm, jnp.zeros((pad,), jnp.float32)])
    zinit = jnp.zeros((n, h), jnp.float32)

    coeff_rows = jnp.repeat(cheb_coeffs, c, axis=0).T
    gs_row = jnp.repeat(group_scale, c).reshape(1, h)
    gb_row = jnp.repeat(group_bias, c).reshape(1, h)
    nw_row = norm_weight.reshape(1, h)

    propagate_sc = _build_propagate(n, h, chunks_per_tile)

    def propagate(hx):
        return propagate_sc(hx, src_p, dst_p, w_p, zinit)

    parts = propagate(x)
    t1 = _combine_first(parts, n, h)
    ts = [x, t1]
    tprev2, tprev1 = x, t1
    for _k in range(2, kmax):
        parts = propagate(tprev1)
        tk = _combine_step(parts, tprev2, n, h)
        ts.append(tk)
        tprev2, tprev1 = tprev1, tk
    parts_last = propagate(tprev1)
    return _final_tail(parts_last, ts, coeff_rows, gs_row, gb_row, nw_row,
                       n, h)


# speedup vs baseline: 1.3230x; 1.0163x over previous
import dataclasses
import functools

import jax
import jax.numpy as jnp
from jax import lax
from jax.experimental import pallas as pl
from jax.experimental.pallas import tpu as pltpu
from jax.experimental.pallas import tpu_sc as plsc

_NUM_CORES = 2
_NUM_SUBCORES = 16
_NTILES = _NUM_CORES * _NUM_SUBCORES
_CH = 112


def _build_propagate(n, h, chunks_per_tile):
    rows_per_tile = (n // _NUM_SUBCORES) // 8 * 8
    rows_last = n - rows_per_tile * (_NUM_SUBCORES - 1)
    mesh = plsc.VectorSubcoreMesh(
        core_axis_name="c", subcore_axis_name="s", num_cores=_NUM_CORES,
        num_subcores=_NUM_SUBCORES)
    cp = pltpu.CompilerParams()
    if "needs_layout_passes" in pltpu.CompilerParams.__dataclass_fields__:
        cp = dataclasses.replace(cp, needs_layout_passes=False)

    @functools.partial(
        pl.kernel,
        out_type=jax.ShapeDtypeStruct((_NUM_CORES, n, h), jnp.float32),
        mesh=mesh,
        scratch_types=[
            pltpu.VMEM_SHARED((n, h), jnp.float32),
            pltpu.VMEM((3, _CH), jnp.int32),
            pltpu.VMEM((6, _CH), jnp.int32),
            pltpu.VMEM((3, _CH), jnp.float32),
            pltpu.VMEM((_CH, h), jnp.float32),
            pltpu.VMEM((_CH, h), jnp.float32),
            pltpu.VMEM((_CH, h), jnp.float32),
        ] + [pltpu.SemaphoreType.DMA] * 12,
        compiler_params=cp,
    )
    def propagate_sc(hx_hbm, src_hbm, dst_hbm, w_hbm, zinit_hbm, part_hbm,
                     acc_sh, sb, db, wb, rows0, rows1, rows2, *sems):
        ci = lax.axis_index("c")
        si = lax.axis_index("s")
        tile = ci * _NUM_SUBCORES + si
        chunk0 = tile * chunks_per_tile

        r0 = pl.multiple_of(si * rows_per_tile, 8)

        dnums = lax.GatherDimensionNumbers(
            offset_dims=(), collapsed_slice_dims=(0,), start_index_map=(0,))

        def splat(vec, e):
            idx = jnp.full((16, 1), e, jnp.int32)
            return lax.gather(vec, idx, dnums, slice_sizes=(1,),
                              mode=lax.GatherScatterMode.PROMISE_IN_BOUNDS)

        def scale(rbuf, p):
            @pl.loop(0, _CH // 16)
            def _grp(gi):
                wv16 = wb[p, pl.ds(gi * 16, 16)]
                for e in range(16):
                    ei = gi * 16 + e
                    wsp = splat(wv16, e)
                    r = rbuf.at[ei]
                    for j in range(h // 16):
                        r[pl.ds(j * 16, 16)] = r[pl.ds(j * 16, 16)] * wsp

        rows = (rows0, rows1, rows2)
        gsem = sems[0:3]
        ssem = sems[3:6]
        esem = sems[6:12]
        nc = chunks_per_tile

        def eslice(arr, t):
            off = pl.multiple_of((chunk0 + t) * _CH, 8)
            return arr.at[pl.ds(off, _CH)]

        def start_eloads(t, i):
            pltpu.async_copy(eslice(src_hbm, t), sb.at[i % 3], esem[i % 6])
            pltpu.async_copy(eslice(dst_hbm, t), db.at[i % 6], esem[i % 6])
            pltpu.async_copy(eslice(w_hbm, t), wb.at[i % 3], esem[i % 6])

        def wait_eloads(t, i):
            sem = esem[i % 6]
            pltpu.make_async_copy(eslice(src_hbm, t), sb.at[i % 3], sem).wait()
            pltpu.make_async_copy(eslice(dst_hbm, t), db.at[i % 6], sem).wait()
            pltpu.make_async_copy(eslice(w_hbm, t), wb.at[i % 3], sem).wait()

        def start_gather(i):
            pltpu.async_copy(hx_hbm.at[sb.at[i % 3]], rows[i % 3],
                             gsem[i % 3])

        def wait_gather(i):
            pltpu.make_async_copy(hx_hbm.at[sb.at[i % 3]], rows[i % 3],
                                  gsem[i % 3]).wait()

        def start_scatter(i):
            pltpu.async_copy(rows[i % 3], acc_sh.at[db.at[i % 6]],
                             ssem[i % 3], add=True)

        def wait_scatter(i):
            pltpu.make_async_copy(rows[i % 3], acc_sh.at[db.at[i % 6]],
                                  ssem[i % 3]).wait()

        start_eloads(0, 0)
        start_eloads(1, 1)
        wait_eloads(0, 0)
        start_gather(0)

        @pl.when(si < _NUM_SUBCORES - 1)
        def _():
            pltpu.sync_copy(zinit_hbm.at[pl.ds(r0, rows_per_tile)],
                            acc_sh.at[pl.ds(r0, rows_per_tile)])

        @pl.when(si == _NUM_SUBCORES - 1)
        def _():
            pltpu.sync_copy(zinit_hbm.at[pl.ds(r0, rows_last)],
                            acc_sh.at[pl.ds(r0, rows_last)])

        plsc.subcore_barrier()

        @pl.loop(0, nc, step=6)
        def _hex(t):
            for i in range(6):
                c = t + i

                @pl.when(c >= 2)
                def _():
                    wait_scatter(i + 4)

                @pl.when(c + 1 < nc)
                def _():
                    wait_eloads(c + 1, i + 1)
                    start_gather(i + 1)

                wait_gather(i)
                scale(rows[i % 3], i % 3)
                start_scatter(i)

                @pl.when(c + 2 < nc)
                def _():
                    start_eloads(c + 2, i + 2)

        wait_scatter(nc - 2)
        wait_scatter(nc - 1)

        plsc.subcore_barrier()

        @pl.when(si < _NUM_SUBCORES - 1)
        def _():
            pltpu.sync_copy(acc_sh.at[pl.ds(r0, rows_per_tile)],
                            part_hbm.at[ci].at[pl.ds(r0, rows_per_tile)])

        @pl.when(si == _NUM_SUBCORES - 1)
        def _():
            pltpu.sync_copy(acc_sh.at[pl.ds(r0, rows_last)],
                            part_hbm.at[ci].at[pl.ds(r0, rows_last)])

    return propagate_sc


def _combine_first(parts, n, h):
    def body(a_ref, o_ref):
        o_ref[...] = a_ref[0] + a_ref[1]
    return pl.pallas_call(
        body, out_shape=jax.ShapeDtypeStruct((n, h), jnp.float32))(parts)


def _combine_step(parts, tprev2, n, h):
    def body(a_ref, t_ref, o_ref):
        o_ref[...] = 2.0 * (a_ref[0] + a_ref[1]) - t_ref[...]
    return pl.pallas_call(
        body, out_shape=jax.ShapeDtypeStruct((n, h), jnp.float32))(parts, tprev2)


def _final_tail(parts_last, ts, coeff_rows, gs_row, gb_row, nw_row, n, h):
    kmax = coeff_rows.shape[0] - 1
    eps = jnp.finfo(jnp.float32).eps

    def body(*refs):
        t_refs = refs[:kmax]
        a_ref = refs[kmax]
        coef_ref = refs[kmax + 1]
        gs_ref, gb_ref, nw_ref = refs[kmax + 2:kmax + 5]
        o_ref = refs[kmax + 5]
        t_last = 2.0 * (a_ref[0] + a_ref[1]) - t_refs[kmax - 2][...]
        res = coef_ref[kmax:kmax + 1] * t_last
        for k in range(kmax):
            res = res + coef_ref[k:k + 1] * t_refs[k][...]
        res = res * gs_ref[...] + gb_ref[...]
        ms = jnp.mean(res * res, axis=-1, keepdims=True)
        y = res * lax.rsqrt(ms + eps) * nw_ref[...]
        o_ref[...] = y * jax.nn.sigmoid(y)

    return pl.pallas_call(
        body, out_shape=jax.ShapeDtypeStruct((n, h), jnp.float32))(
            *ts, parts_last, coeff_rows, gs_row, gb_row, nw_row)


def kernel(x, edge_index, edge_weight_norm, cheb_coeffs, group_scale,
           group_bias, norm_weight):
    n, h = x.shape
    e = edge_index.shape[1]
    g = group_scale.shape[0]
    c = h // g
    kmax = cheb_coeffs.shape[1] - 1

    chunks_per_tile = -(-e // (_CH * _NTILES))
    chunks_per_tile = -(-chunks_per_tile // 6) * 6
    e_pad = chunks_per_tile * _CH * _NTILES
    pad = e_pad - e

    pad_idx = jnp.arange(pad, dtype=jnp.int32) % n
    src_p = jnp.concatenate([edge_index[0], pad_idx])
    dst_p = jnp.concatenate([edge_index[1], pad_idx])
    w_p = jnp.concatenate([edge_weight_norm, jnp.zeros((pad,), jnp.float32)])
    zinit = jnp.zeros((n, h), jnp.float32)

    coeff_rows = jnp.repeat(cheb_coeffs, c, axis=0).T
    gs_row = jnp.repeat(group_scale, c).reshape(1, h)
    gb_row = jnp.repeat(group_bias, c).reshape(1, h)
    nw_row = norm_weight.reshape(1, h)

    propagate_sc = _build_propagate(n, h, chunks_per_tile)

    def propagate(hx):
        return propagate_sc(hx, src_p, dst_p, w_p, zinit)

    parts = propagate(x)
    t1 = _combine_first(parts, n, h)
    ts = [x, t1]
    tprev2, tprev1 = x, t1
    for _k in range(2, kmax):
        parts = propagate(tprev1)
        tk = _combine_step(parts, tprev2, n, h)
        ts.append(tk)
        tprev2, tprev1 = tprev1, tk
    parts_last = propagate(tprev1)
    return _final_tail(parts_last, ts, coeff_rows, gs_row, gb_row, nw_row,
                       n, h)
